# baseline (speedup 1.0000x reference)
import jax
import jax.numpy as jnp
from jax import lax
from jax.experimental import pallas as pl
from jax.experimental.pallas import tpu as pltpu
from jax.experimental.pallas import tpu_sc as plsc

_B = 1638400
_V = 1000000
_NC = 2
_NS = 16
_NW = _NC * _NS
_BPW = _B // _NW
_L = 16

_CH = 62496
_CH_LAST = _V - 15 * _CH
_SCH = 6944
_SNJ = _CH // _SCH
_SCH_LAST = 6256
_SNJ_LAST = _CH_LAST // _SCH_LAST
_SHALF = 7680

_NG = 10
_GS = 3440
_GH = 1680
_SPLIT = _NG * _GS
_BUF = 3 * (_GS + _GH)


def _landmark_body(counts_hbm, obs_hbm, idx_hbm, out_hbm, idx_v, buf_v,
                   obs_v, table_sh, isem, ssem, gs0, gs1, gs2, gh0, gh1,
                   gh2, os0, os1, os2, oh0, oh1, oh2):
    c = lax.axis_index("c")
    s = lax.axis_index("s")
    wid = s * _NC + c
    base = wid * _BPW

    def sbuf(b):
        return buf_v.at[pl.ds(b * _GS, _GS)]

    def hbuf(b):
        return buf_v.at[pl.ds(3 * _GS + b * _GH, _GH)]

    icp = pltpu.async_copy(idx_hbm.at[pl.ds(base, _BPW)], idx_v, isem)
    pltpu.sync_copy(obs_hbm, obs_v.at[pl.ds(0, 1)])
    recip = (1.0 / obs_v[pl.ds(0, _L)])[0]
    off = s * _CH
    gsem = (gs0, gs1, gs2)
    osem = (os0, os1, os2)
    hsem = (gh0, gh1, gh2)
    hosem = (oh0, oh1, oh2)

    def stage(sch, snj):
        si = [None, None]
        so = [None, None]
        for j in range(snj):
            b = j % 2
            if j >= 2:
                so[b].wait()
            si[b] = pltpu.async_copy(
                counts_hbm.at[pl.ds(off + j * sch, sch)],
                buf_v.at[pl.ds(b * _SHALF, sch)], gsem[b])
            if j >= 1:
                pb = 1 - b
                si[pb].wait()
                so[pb] = pltpu.async_copy(
                    buf_v.at[pl.ds(pb * _SHALF, sch)],
                    table_sh.at[pl.ds(off + (j - 1) * sch, sch)], osem[pb])
        lb = (snj - 1) % 2
        si[lb].wait()
        so[1 - lb].wait()
        pltpu.async_copy(
            buf_v.at[pl.ds(lb * _SHALF, sch)],
            table_sh.at[pl.ds(off + (snj - 1) * sch, sch)], ssem).wait()

    with jax.named_scope("stage_table"):
        @pl.when(s < _NS - 1)
        def _():
            stage(_SCH, _SNJ)

        @pl.when(s == _NS - 1)
        def _():
            stage(_SCH_LAST, _SNJ_LAST)

        icp.wait()
    with jax.named_scope("stage_barrier"):
        plsc.subcore_barrier()

    def scale_words(word_off, n_iter):
        def body(i, carry):
            sl = pl.ds(word_off + i * _L, _L)
            buf_v[sl] = buf_v[sl] * recip
            return carry
        lax.fori_loop(0, n_iter, body, 0)

    gS = [None, None, None]
    oS = [None, None, None]
    gH = [None, None, None]
    oH = [None, None, None]

    def process_s(p):
        pb = p % 3
        gS[pb].wait()
        scale_words(pb * _GS, _GS // _L)
        oS[pb] = pltpu.async_copy(
            sbuf(pb), out_hbm.at[pl.ds(base + p * _GS, _GS)], osem[pb])

    def process_h(p):
        pb = p % 3
        gH[pb].wait()
        scale_words(3 * _GS + pb * _GH, _GH // _L)
        oH[pb] = pltpu.async_copy(
            hbuf(pb), out_hbm.at[pl.ds(base + _SPLIT + p * _GH, _GH)],
            hosem[pb])

    with jax.named_scope("gather_loop"):
        for j in range(_NG):
            b = j % 3
            if j >= 3:
                oS[b].wait()
                oH[b].wait()
            gS[b] = pltpu.async_copy(
                table_sh.at[idx_v.at[pl.ds(j * _GS, _GS)]], sbuf(b), gsem[b])
            gH[b] = pltpu.async_copy(
                counts_hbm.at[idx_v.at[pl.ds(_SPLIT + j * _GH, _GH)]],
                hbuf(b), hsem[b])
            if j >= 2:
                process_s(j - 2)
                process_h(j - 2)

    with jax.named_scope("gather_drain"):
        process_s(_NG - 2)
        process_h(_NG - 2)
        process_s(_NG - 1)
        process_h(_NG - 1)
        for b in range(3):
            oS[b].wait()
            oH[b].wait()


def kernel(counts, obs_count, landmark_indices):
    mesh = plsc.VectorSubcoreMesh(core_axis_name="c", subcore_axis_name="s")
    k = pl.kernel(
        _landmark_body,
        mesh=mesh,
        out_type=jax.ShapeDtypeStruct((_B,), jnp.float32),
        scratch_types=[
            pltpu.VMEM((_BPW,), jnp.int32),
            pltpu.VMEM((_BUF,), jnp.float32),
            pltpu.VMEM((_L,), jnp.float32),
            pltpu.VMEM_SHARED((_V,), jnp.float32),
        ] + [pltpu.SemaphoreType.DMA] * 14,
    )
    return k(counts, obs_count, landmark_indices)


# speedup vs baseline: 1.0093x; 1.0093x over previous
import jax
import jax.numpy as jnp
from jax import lax
from jax.experimental import pallas as pl
from jax.experimental.pallas import tpu as pltpu
from jax.experimental.pallas import tpu_sc as plsc

_B = 1638400
_V = 1000000
_NC = 2
_NS = 16
_NW = _NC * _NS
_BPW = _B // _NW
_L = 16

_CH = 62496
_CH_LAST = _V - 15 * _CH
_SCH = 6944
_SNJ = _CH // _SCH
_SCH_LAST = 6256
_SNJ_LAST = _CH_LAST // _SCH_LAST
_SHALF = 7680

_NG = 10
_GS = 3520
_GH = 1600
_SPLIT = _NG * _GS
_BUF = 3 * (_GS + _GH)


def _landmark_body(counts_hbm, obs_hbm, idx_hbm, out_hbm, idx_v, buf_v,
                   obs_v, table_sh, isem, ssem, gs0, gs1, gs2, gh0, gh1,
                   gh2, os0, os1, os2, oh0, oh1, oh2):
    c = lax.axis_index("c")
    s = lax.axis_index("s")
    wid = s * _NC + c
    base = wid * _BPW

    def sbuf(b):
        return buf_v.at[pl.ds(b * _GS, _GS)]

    def hbuf(b):
        return buf_v.at[pl.ds(3 * _GS + b * _GH, _GH)]

    icp = pltpu.async_copy(idx_hbm.at[pl.ds(base, _BPW)], idx_v, isem)
    pltpu.sync_copy(obs_hbm, obs_v.at[pl.ds(0, 1)])
    recip = (1.0 / obs_v[pl.ds(0, _L)])[0]
    off = s * _CH
    gsem = (gs0, gs1, gs2)
    osem = (os0, os1, os2)
    hsem = (gh0, gh1, gh2)
    hosem = (oh0, oh1, oh2)

    def stage(sch, snj):
        si = [None, None]
        so = [None, None]
        for j in range(snj):
            b = j % 2
            if j >= 2:
                so[b].wait()
            si[b] = pltpu.async_copy(
                counts_hbm.at[pl.ds(off + j * sch, sch)],
                buf_v.at[pl.ds(b * _SHALF, sch)], gsem[b])
            if j >= 1:
                pb = 1 - b
                si[pb].wait()
                so[pb] = pltpu.async_copy(
                    buf_v.at[pl.ds(pb * _SHALF, sch)],
                    table_sh.at[pl.ds(off + (j - 1) * sch, sch)], osem[pb])
        lb = (snj - 1) % 2
        si[lb].wait()
        so[1 - lb].wait()
        pltpu.async_copy(
            buf_v.at[pl.ds(lb * _SHALF, sch)],
            table_sh.at[pl.ds(off + (snj - 1) * sch, sch)], ssem).wait()

    with jax.named_scope("stage_table"):
        @pl.when(s < _NS - 1)
        def _():
            stage(_SCH, _SNJ)

        @pl.when(s == _NS - 1)
        def _():
            stage(_SCH_LAST, _SNJ_LAST)

        icp.wait()
    with jax.named_scope("stage_barrier"):
        plsc.subcore_barrier()

    def scale_words(word_off, n_iter):
        def body(i, carry):
            sl = pl.ds(word_off + i * _L, _L)
            buf_v[sl] = buf_v[sl] * recip
            return carry
        lax.fori_loop(0, n_iter, body, 0)

    gS = [None, None, None]
    oS = [None, None, None]
    gH = [None, None, None]
    oH = [None, None, None]

    def process_s(p):
        pb = p % 3
        gS[pb].wait()
        scale_words(pb * _GS, _GS // _L)
        oS[pb] = pltpu.async_copy(
            sbuf(pb), out_hbm.at[pl.ds(base + p * _GS, _GS)], osem[pb])

    def process_h(p):
        pb = p % 3
        gH[pb].wait()
        scale_words(3 * _GS + pb * _GH, _GH // _L)
        oH[pb] = pltpu.async_copy(
            hbuf(pb), out_hbm.at[pl.ds(base + _SPLIT + p * _GH, _GH)],
            hosem[pb])

    with jax.named_scope("gather_loop"):
        for j in range(_NG):
            b = j % 3
            if j >= 3:
                oS[b].wait()
                oH[b].wait()
            gS[b] = pltpu.async_copy(
                table_sh.at[idx_v.at[pl.ds(j * _GS, _GS)]], sbuf(b), gsem[b])
            gH[b] = pltpu.async_copy(
                counts_hbm.at[idx_v.at[pl.ds(_SPLIT + j * _GH, _GH)]],
                hbuf(b), hsem[b])
            if j >= 2:
                process_s(j - 2)
                process_h(j - 2)

    with jax.named_scope("gather_drain"):
        process_s(_NG - 2)
        process_h(_NG - 2)
        process_s(_NG - 1)
        process_h(_NG - 1)
        for b in range(3):
            oS[b].wait()
            oH[b].wait()


def kernel(counts, obs_count, landmark_indices):
    mesh = plsc.VectorSubcoreMesh(core_axis_name="c", subcore_axis_name="s")
    k = pl.kernel(
        _landmark_body,
        mesh=mesh,
        out_type=jax.ShapeDtypeStruct((_B,), jnp.float32),
        scratch_types=[
            pltpu.VMEM((_BPW,), jnp.int32),
            pltpu.VMEM((_BUF,), jnp.float32),
            pltpu.VMEM((_L,), jnp.float32),
            pltpu.VMEM_SHARED((_V,), jnp.float32),
        ] + [pltpu.SemaphoreType.DMA] * 14,
    )
    return k(counts, obs_count, landmark_indices)
